# Initial kernel scaffold; baseline (speedup 1.0000x reference)
import functools

import jax
import jax.numpy as jnp
from jax import lax
from jax.experimental import pallas as pl
from jax.experimental.pallas import tpu as pltpu, tpu_sc as plsc

N_NODES = 10000
N_EDGES = 160000
D_NODE = 256
D_EDGE = 64
HIDDEN = 256
N_EXPERTS = 3

SC_CORES = 2
SC_SUBCORES = 16
SC_WORKERS = SC_CORES * SC_SUBCORES

CHUNK = 128
N_CHUNKS = N_EDGES // CHUNK
BE = 640
NB = N_EDGES // BE
NODE_BLK = 200


def _proj_body(n_ref, w_ref, o_ref):
    o_ref[...] = jnp.dot(n_ref[...], w_ref[0],
                         preferred_element_type=jnp.float32)[None]


def _project_nodes(node_embed, w_stacked):
    return pl.pallas_call(
        _proj_body,
        grid=(2 * N_EXPERTS, N_NODES // NODE_BLK),
        in_specs=[
            pl.BlockSpec((NODE_BLK, D_NODE), lambda j, i: (i, 0)),
            pl.BlockSpec((1, D_NODE, HIDDEN), lambda j, i: (j, 0, 0)),
        ],
        out_specs=pl.BlockSpec((1, NODE_BLK, HIDDEN), lambda j, i: (j, i, 0)),
        out_shape=jax.ShapeDtypeStruct((2 * N_EXPERTS, N_NODES, HIDDEN),
                                       jnp.float32),
    )(node_embed, w_stacked)


def _sc_gather_body(table_hbm, src_hbm, dst_hbm, type_hbm,
                    hs_hbm, hd_hbm,
                    srcb, dstb, typeb, idxs, idxd, rows_s, rows_d,
                    sem_s, sem_d):
    wid = lax.axis_index("s") * SC_CORES + lax.axis_index("c")
    n_mine = (N_CHUNKS - wid + SC_WORKERS - 1) // SC_WORKERS

    def chunk_body(c, carry):
        base = (wid + c * SC_WORKERS) * CHUNK
        pltpu.sync_copy(src_hbm.at[pl.ds(base, CHUNK)], srcb)
        pltpu.sync_copy(dst_hbm.at[pl.ds(base, CHUNK)], dstb)
        pltpu.sync_copy(type_hbm.at[pl.ds(base, CHUNK)], typeb)
        for i in range(CHUNK // 16):
            sl = pl.ds(i * 16, 16)
            t16 = typeb[sl] * N_NODES
            idxs[sl] = t16 + srcb[sl]
            idxd[sl] = t16 + dstb[sl] + N_EXPERTS * N_NODES
        cp_s = pltpu.async_copy(table_hbm.at[idxs], rows_s, sem_s)
        cp_d = pltpu.async_copy(table_hbm.at[idxd], rows_d, sem_d)
        cp_s.wait()
        cp_d.wait()
        pltpu.sync_copy(rows_s, hs_hbm.at[pl.ds(base, CHUNK)])
        pltpu.sync_copy(rows_d, hd_hbm.at[pl.ds(base, CHUNK)])
        return carry

    lax.fori_loop(0, n_mine, chunk_body, 0)


def _sc_gather(table, src, dst, etype):
    mesh = plsc.VectorSubcoreMesh(core_axis_name="c", subcore_axis_name="s")
    kern = pl.kernel(
        _sc_gather_body,
        out_type=(
            jax.ShapeDtypeStruct((N_EDGES, HIDDEN), jnp.float32),
            jax.ShapeDtypeStruct((N_EDGES, HIDDEN), jnp.float32),
        ),
        mesh=mesh,
        scratch_types=[
            pltpu.VMEM((CHUNK,), jnp.int32),
            pltpu.VMEM((CHUNK,), jnp.int32),
            pltpu.VMEM((CHUNK,), jnp.int32),
            pltpu.VMEM((CHUNK,), jnp.int32),
            pltpu.VMEM((CHUNK,), jnp.int32),
            pltpu.VMEM((CHUNK, HIDDEN), jnp.float32),
            pltpu.VMEM((CHUNK, HIDDEN), jnp.float32),
            pltpu.SemaphoreType.DMA,
            pltpu.SemaphoreType.DMA,
        ],
    )
    return kern(table, src, dst, etype)


def _edge_body(hs_ref, hd_ref, es_ref, t_ref, bz_ref,
               w1c_ref, b1_ref, w2_ref, b2_ref, w3_ref, b3_ref,
               dz_ref, rho_ref):
    t = t_ref[0, 0]
    hsum = hs_ref[...] + hd_ref[...]
    es = es_ref[...]

    pre = jnp.zeros((BE, HIDDEN), jnp.float32)
    for e in range(N_EXPERTS):
        pe = jnp.dot(es, w1c_ref[e], preferred_element_type=jnp.float32)
        pe = pe + b1_ref[e][None, :]
        pre = pre + jnp.where((t == e)[:, None], pe, 0.0)
    h1 = jax.nn.gelu(hsum + pre, approximate=False)

    delta = jnp.zeros((BE,), jnp.float32)
    for e in range(N_EXPERTS):
        g = jnp.dot(h1, w2_ref[e], preferred_element_type=jnp.float32)
        g = jax.nn.gelu(g + b2_ref[e][None, :], approximate=False)
        z = jnp.sum(g * w3_ref[e], axis=1) + b3_ref[e, 0]
        delta = delta + jnp.where(t == e, z, 0.0)

    dz_ref[0, 0] = delta
    rho_ref[0, 0] = jnp.tanh(bz_ref[0, 0] + delta)


def _edge_mlp(hs, hd, edge_state, t3, bz3, w1c, b1, w2, b2, w3t, b3b):
    full = lambda s: pl.BlockSpec(s, lambda i: tuple(0 for _ in s))
    return pl.pallas_call(
        _edge_body,
        grid=(NB,),
        in_specs=[
            pl.BlockSpec((BE, HIDDEN), lambda i: (i, 0)),
            pl.BlockSpec((BE, HIDDEN), lambda i: (i, 0)),
            pl.BlockSpec((BE, D_EDGE), lambda i: (i, 0)),
            pl.BlockSpec((1, 1, BE), lambda i: (i, 0, 0)),
            pl.BlockSpec((1, 1, BE), lambda i: (i, 0, 0)),
            full((N_EXPERTS, D_EDGE, HIDDEN)),
            full((N_EXPERTS, HIDDEN)),
            full((N_EXPERTS, HIDDEN, HIDDEN)),
            full((N_EXPERTS, HIDDEN)),
            full((N_EXPERTS, 1, HIDDEN)),
            full((N_EXPERTS, 1, BE)),
        ],
        out_specs=[
            pl.BlockSpec((1, 1, BE), lambda i: (i, 0, 0)),
            pl.BlockSpec((1, 1, BE), lambda i: (i, 0, 0)),
        ],
        out_shape=[
            jax.ShapeDtypeStruct((NB, 1, BE), jnp.float32),
            jax.ShapeDtypeStruct((NB, 1, BE), jnp.float32),
        ],
    )(hs, hd, edge_state, t3, bz3, w1c, b1, w2, b2, w3t, b3b)


def kernel(node_embed, edge_state, edge_index, edge_type, baseline_z,
           W1, b1, W2, b2, W3, b3):
    src = edge_index[0].astype(jnp.int32)
    dst = edge_index[1].astype(jnp.int32)
    etype = edge_type.astype(jnp.int32)

    w_stacked = jnp.concatenate([W1[:, :D_NODE, :], W1[:, D_NODE:2 * D_NODE, :]],
                                axis=0)
    table = _project_nodes(node_embed, w_stacked).reshape(
        2 * N_EXPERTS * N_NODES, HIDDEN)

    hs, hd = _sc_gather(table, src, dst, etype)

    w1c = W1[:, 2 * D_NODE:, :]
    w3t = W3.transpose(0, 2, 1)
    b3b = jnp.broadcast_to(b3.reshape(N_EXPERTS, 1, 1), (N_EXPERTS, 1, BE))
    t3 = etype.reshape(NB, 1, BE)
    bz3 = baseline_z.reshape(NB, 1, BE)

    dz3, rho3 = _edge_mlp(hs, hd, edge_state, t3, bz3,
                          w1c, b1, W2, b2, w3t, b3b)
    return dz3.reshape(N_EDGES), rho3.reshape(N_EDGES)


# speedup vs baseline: 2.1516x; 2.1516x over previous
import functools

import jax
import jax.numpy as jnp
from jax import lax
from jax.experimental import pallas as pl
from jax.experimental.pallas import tpu as pltpu, tpu_sc as plsc

N_NODES = 10000
N_EDGES = 160000
D_NODE = 256
D_EDGE = 64
HIDDEN = 256
N_EXPERTS = 3

SC_CORES = 2
SC_SUBCORES = 16
SC_WORKERS = SC_CORES * SC_SUBCORES

CHUNK = 128
CHUNKS_PER_TILE = 40
E_PAD = 32 * CHUNKS_PER_TILE * CHUNK
EDGES_PER_TILE = CHUNKS_PER_TILE * CHUNK
BE = 640
NB = N_EDGES // BE
NODE_BLK = 200


def _gelu_exact(x):
    return 0.5 * x * (1.0 + lax.erf(x * 0.7071067811865476))


def _proj_body(n_ref, w_ref, o_ref):
    o_ref[...] = jnp.dot(n_ref[...], w_ref[0],
                         preferred_element_type=jnp.float32)[None]


def _project_nodes(node_embed, w_stacked):
    return pl.pallas_call(
        _proj_body,
        grid=(2 * N_EXPERTS, N_NODES // NODE_BLK),
        in_specs=[
            pl.BlockSpec((NODE_BLK, D_NODE), lambda j, i: (i, 0)),
            pl.BlockSpec((1, D_NODE, HIDDEN), lambda j, i: (j, 0, 0)),
        ],
        out_specs=pl.BlockSpec((1, NODE_BLK, HIDDEN), lambda j, i: (j, i, 0)),
        out_shape=jax.ShapeDtypeStruct((2 * N_EXPERTS, N_NODES, HIDDEN),
                                       jnp.float32),
    )(node_embed, w_stacked)


def _sc_gather_body(table_hbm, src_hbm, dst_hbm, type_hbm,
                    hs_hbm, hd_hbm,
                    srcb, dstb, typeb, idxs, idxd, rows_s, rows_d,
                    sem_s, sem_d):
    wid = lax.axis_index("s") * SC_CORES + lax.axis_index("c")
    tile_base = wid * EDGES_PER_TILE

    pltpu.sync_copy(src_hbm.at[pl.ds(tile_base, EDGES_PER_TILE)], srcb)
    pltpu.sync_copy(dst_hbm.at[pl.ds(tile_base, EDGES_PER_TILE)], dstb)
    pltpu.sync_copy(type_hbm.at[pl.ds(tile_base, EDGES_PER_TILE)], typeb)

    def idx_body(i, carry):
        sl = pl.ds(i * 16, 16)
        t16 = typeb[sl] * N_NODES
        idxs[sl] = t16 + srcb[sl]
        idxd[sl] = t16 + dstb[sl] + N_EXPERTS * N_NODES
        return carry

    lax.fori_loop(0, EDGES_PER_TILE // 16, idx_body, 0)

    def chunk_body(c, carry):
        base = tile_base + c * CHUNK
        csl = pl.ds(c * CHUNK, CHUNK)
        cp_s = pltpu.async_copy(table_hbm.at[idxs.at[csl]], rows_s, sem_s)
        cp_d = pltpu.async_copy(table_hbm.at[idxd.at[csl]], rows_d, sem_d)
        cp_s.wait()
        cp_d.wait()
        pltpu.sync_copy(rows_s, hs_hbm.at[pl.ds(base, CHUNK)])
        pltpu.sync_copy(rows_d, hd_hbm.at[pl.ds(base, CHUNK)])
        return carry

    lax.fori_loop(0, CHUNKS_PER_TILE, chunk_body, 0)


def _sc_gather(table, src, dst, etype):
    mesh = plsc.VectorSubcoreMesh(core_axis_name="c", subcore_axis_name="s")
    kern = pl.kernel(
        _sc_gather_body,
        out_type=(
            jax.ShapeDtypeStruct((E_PAD, HIDDEN), jnp.float32),
            jax.ShapeDtypeStruct((E_PAD, HIDDEN), jnp.float32),
        ),
        mesh=mesh,
        scratch_types=[
            pltpu.VMEM((EDGES_PER_TILE,), jnp.int32),
            pltpu.VMEM((EDGES_PER_TILE,), jnp.int32),
            pltpu.VMEM((EDGES_PER_TILE,), jnp.int32),
            pltpu.VMEM((EDGES_PER_TILE,), jnp.int32),
            pltpu.VMEM((EDGES_PER_TILE,), jnp.int32),
            pltpu.VMEM((CHUNK, HIDDEN), jnp.float32),
            pltpu.VMEM((CHUNK, HIDDEN), jnp.float32),
            pltpu.SemaphoreType.DMA,
            pltpu.SemaphoreType.DMA,
        ],
    )
    return kern(table, src, dst, etype)


def _edge_body(hs_ref, hd_ref, es_ref, t_ref, bz_ref,
               w1c_ref, b1_ref, w2_ref, b2_ref, w3_ref, b3_ref,
               dz_ref, rho_ref):
    t = t_ref[0, 0]
    hsum = hs_ref[...] + hd_ref[...]
    es = es_ref[...]

    delta = jnp.zeros((BE,), jnp.float32)
    for e in range(N_EXPERTS):
        pe = jnp.dot(es, w1c_ref[e], preferred_element_type=jnp.float32)
        h1 = _gelu_exact(hsum + pe + b1_ref[e][None, :])
        g = jnp.dot(h1, w2_ref[e], preferred_element_type=jnp.float32)
        g = _gelu_exact(g + b2_ref[e][None, :])
        z = jnp.sum(g * w3_ref[e], axis=1) + b3_ref[e, 0]
        delta = delta + jnp.where(t == e, z, 0.0)

    dz_ref[0, 0] = delta
    rho_ref[0, 0] = jnp.tanh(bz_ref[0, 0] + delta)


def _edge_mlp(hs, hd, edge_state, t3, bz3, w1c, b1, w2, b2, w3t, b3b):
    full = lambda s: pl.BlockSpec(s, lambda i: tuple(0 for _ in s))
    return pl.pallas_call(
        _edge_body,
        grid=(NB,),
        in_specs=[
            pl.BlockSpec((BE, HIDDEN), lambda i: (i, 0)),
            pl.BlockSpec((BE, HIDDEN), lambda i: (i, 0)),
            pl.BlockSpec((BE, D_EDGE), lambda i: (i, 0)),
            pl.BlockSpec((1, 1, BE), lambda i: (i, 0, 0)),
            pl.BlockSpec((1, 1, BE), lambda i: (i, 0, 0)),
            full((N_EXPERTS, D_EDGE, HIDDEN)),
            full((N_EXPERTS, HIDDEN)),
            full((N_EXPERTS, HIDDEN, HIDDEN)),
            full((N_EXPERTS, HIDDEN)),
            full((N_EXPERTS, 1, HIDDEN)),
            full((N_EXPERTS, 1, BE)),
        ],
        out_specs=[
            pl.BlockSpec((1, 1, BE), lambda i: (i, 0, 0)),
            pl.BlockSpec((1, 1, BE), lambda i: (i, 0, 0)),
        ],
        out_shape=[
            jax.ShapeDtypeStruct((NB, 1, BE), jnp.float32),
            jax.ShapeDtypeStruct((NB, 1, BE), jnp.float32),
        ],
    )(hs, hd, edge_state, t3, bz3, w1c, b1, w2, b2, w3t, b3b)


def kernel(node_embed, edge_state, edge_index, edge_type, baseline_z,
           W1, b1, W2, b2, W3, b3):
    pad = (0, E_PAD - N_EDGES)
    src = jnp.pad(edge_index[0].astype(jnp.int32), pad)
    dst = jnp.pad(edge_index[1].astype(jnp.int32), pad)
    etype = edge_type.astype(jnp.int32)
    etype_p = jnp.pad(etype, pad)

    w_stacked = jnp.concatenate([W1[:, :D_NODE, :], W1[:, D_NODE:2 * D_NODE, :]],
                                axis=0)
    table = _project_nodes(node_embed, w_stacked).reshape(
        2 * N_EXPERTS * N_NODES, HIDDEN)

    hs, hd = _sc_gather(table, src, dst, etype_p)

    w1c = W1[:, 2 * D_NODE:, :]
    w3t = W3.transpose(0, 2, 1)
    b3b = jnp.broadcast_to(b3.reshape(N_EXPERTS, 1, 1), (N_EXPERTS, 1, BE))
    t3 = etype.reshape(NB, 1, BE)
    bz3 = baseline_z.reshape(NB, 1, BE)

    dz3, rho3 = _edge_mlp(hs, hd, edge_state, t3, bz3,
                          w1c, b1, W2, b2, w3t, b3b)
    return dz3.reshape(N_EDGES), rho3.reshape(N_EDGES)


# speedup vs baseline: 2.7211x; 1.2647x over previous
import functools

import jax
import jax.numpy as jnp
from jax import lax
from jax.experimental import pallas as pl
from jax.experimental.pallas import tpu as pltpu, tpu_sc as plsc

N_NODES = 10000
N_EDGES = 160000
D_NODE = 256
D_EDGE = 64
HIDDEN = 256
N_EXPERTS = 3

SC_CORES = 2
SC_SUBCORES = 16
SC_WORKERS = SC_CORES * SC_SUBCORES

CHUNK = 128
CHUNKS_PER_TILE = 40
E_PAD = 32 * CHUNKS_PER_TILE * CHUNK
EDGES_PER_TILE = CHUNKS_PER_TILE * CHUNK
BE = 640
NB = N_EDGES // BE
NODE_BLK = 200


def _gelu_exact(x):
    return 0.5 * x * (1.0 + lax.erf(x * 0.7071067811865476))


def _proj_body(n_ref, w_ref, o_ref):
    o_ref[...] = jnp.dot(n_ref[...], w_ref[0],
                         preferred_element_type=jnp.float32)[None]


def _project_nodes(node_embed, w_stacked):
    return pl.pallas_call(
        _proj_body,
        grid=(2 * N_EXPERTS, N_NODES // NODE_BLK),
        in_specs=[
            pl.BlockSpec((NODE_BLK, D_NODE), lambda j, i: (i, 0)),
            pl.BlockSpec((1, D_NODE, HIDDEN), lambda j, i: (j, 0, 0)),
        ],
        out_specs=pl.BlockSpec((1, NODE_BLK, HIDDEN), lambda j, i: (j, i, 0)),
        out_shape=jax.ShapeDtypeStruct((2 * N_EXPERTS, N_NODES, HIDDEN),
                                       jnp.float32),
    )(node_embed, w_stacked)


def _sc_gather_body(table_hbm, src_hbm, dst_hbm, type_hbm,
                    hs_hbm, hd_hbm,
                    srcb, dstb, typeb, idxs, idxd, rows_s, rows_d,
                    sem_s, sem_d):
    wid = lax.axis_index("s") * SC_CORES + lax.axis_index("c")
    tile_base = wid * EDGES_PER_TILE

    pltpu.sync_copy(src_hbm.at[pl.ds(tile_base, EDGES_PER_TILE)], srcb)
    pltpu.sync_copy(dst_hbm.at[pl.ds(tile_base, EDGES_PER_TILE)], dstb)
    pltpu.sync_copy(type_hbm.at[pl.ds(tile_base, EDGES_PER_TILE)], typeb)

    def idx_body(i, carry):
        sl = pl.ds(i * 16, 16)
        t16 = typeb[sl] * N_NODES
        idxs[sl] = t16 + srcb[sl]
        idxd[sl] = t16 + dstb[sl] + N_EXPERTS * N_NODES
        return carry

    lax.fori_loop(0, EDGES_PER_TILE // 16, idx_body, 0)

    def chunk_body(c, carry):
        base = tile_base + c * CHUNK
        csl = pl.ds(c * CHUNK, CHUNK)
        cp_s = pltpu.async_copy(table_hbm.at[idxs.at[csl]], rows_s, sem_s)
        cp_d = pltpu.async_copy(table_hbm.at[idxd.at[csl]], rows_d, sem_d)
        cp_s.wait()
        cp_d.wait()
        pltpu.sync_copy(rows_s, hs_hbm.at[pl.ds(base, CHUNK)])
        pltpu.sync_copy(rows_d, hd_hbm.at[pl.ds(base, CHUNK)])
        return carry

    lax.fori_loop(0, CHUNKS_PER_TILE, chunk_body, 0)


def _sc_gather(table, src, dst, etype):
    mesh = plsc.VectorSubcoreMesh(core_axis_name="c", subcore_axis_name="s")
    kern = pl.kernel(
        _sc_gather_body,
        out_type=(
            jax.ShapeDtypeStruct((E_PAD, HIDDEN), jnp.float32),
            jax.ShapeDtypeStruct((E_PAD, HIDDEN), jnp.float32),
        ),
        mesh=mesh,
        scratch_types=[
            pltpu.VMEM((EDGES_PER_TILE,), jnp.int32),
            pltpu.VMEM((EDGES_PER_TILE,), jnp.int32),
            pltpu.VMEM((EDGES_PER_TILE,), jnp.int32),
            pltpu.VMEM((EDGES_PER_TILE,), jnp.int32),
            pltpu.VMEM((EDGES_PER_TILE,), jnp.int32),
            pltpu.VMEM((CHUNK, HIDDEN), jnp.float32),
            pltpu.VMEM((CHUNK, HIDDEN), jnp.float32),
            pltpu.SemaphoreType.DMA,
            pltpu.SemaphoreType.DMA,
        ],
    )
    return kern(table, src, dst, etype)


def _edge_body(hs_ref, hd_ref, es_ref, t_ref, bz_ref,
               w1c_ref, b1_ref, w2_ref, b2_ref, w3_ref, b3_ref,
               dz_ref, rho_ref):
    t = t_ref[...]
    hsum = hs_ref[...] + hd_ref[...]

    def sel(parts):
        acc = jnp.where(t == 0.0, parts[0], 0.0)
        for e in range(1, N_EXPERTS):
            acc = acc + jnp.where(t == float(e), parts[e], 0.0)
        return acc

    pe = jnp.dot(es_ref[...], w1c_ref[...], preferred_element_type=jnp.float32)
    pe1 = sel([pe[:, e * HIDDEN:(e + 1) * HIDDEN] + b1_ref[e][None, :]
               for e in range(N_EXPERTS)])
    h1 = _gelu_exact(hsum + pe1)

    y = jnp.dot(h1, w2_ref[...], preferred_element_type=jnp.float32)
    y1 = sel([y[:, e * HIDDEN:(e + 1) * HIDDEN] + b2_ref[e][None, :]
              for e in range(N_EXPERTS)])
    g = _gelu_exact(y1)

    z3 = jnp.dot(g, w3_ref[...], preferred_element_type=jnp.float32)
    delta = sel([z3[:, e:e + 1] + b3_ref[e, 0] for e in range(N_EXPERTS)])

    dz_ref[...] = delta
    rho_ref[...] = jnp.tanh(bz_ref[...] + delta)


def _edge_mlp(hs, hd, edge_state, t3, bz3, w1c, b1, w2, b2, w3c, b3):
    full = lambda s: pl.BlockSpec(s, lambda i: tuple(0 for _ in s))
    return pl.pallas_call(
        _edge_body,
        grid=(NB,),
        in_specs=[
            pl.BlockSpec((BE, HIDDEN), lambda i: (i, 0)),
            pl.BlockSpec((BE, HIDDEN), lambda i: (i, 0)),
            pl.BlockSpec((BE, D_EDGE), lambda i: (i, 0)),
            pl.BlockSpec((BE, 1), lambda i: (i, 0)),
            pl.BlockSpec((BE, 1), lambda i: (i, 0)),
            full((D_EDGE, N_EXPERTS * HIDDEN)),
            full((N_EXPERTS, HIDDEN)),
            full((HIDDEN, N_EXPERTS * HIDDEN)),
            full((N_EXPERTS, HIDDEN)),
            full((HIDDEN, N_EXPERTS)),
            full((N_EXPERTS, 1)),
        ],
        out_specs=[
            pl.BlockSpec((BE, 1), lambda i: (i, 0)),
            pl.BlockSpec((BE, 1), lambda i: (i, 0)),
        ],
        out_shape=[
            jax.ShapeDtypeStruct((N_EDGES, 1), jnp.float32),
            jax.ShapeDtypeStruct((N_EDGES, 1), jnp.float32),
        ],
    )(hs, hd, edge_state, t3, bz3, w1c, b1, w2, b2, w3c, b3)


def kernel(node_embed, edge_state, edge_index, edge_type, baseline_z,
           W1, b1, W2, b2, W3, b3):
    pad = (0, E_PAD - N_EDGES)
    src = jnp.pad(edge_index[0].astype(jnp.int32), pad)
    dst = jnp.pad(edge_index[1].astype(jnp.int32), pad)
    etype = edge_type.astype(jnp.int32)
    etype_p = jnp.pad(etype, pad)

    w_stacked = jnp.concatenate([W1[:, :D_NODE, :], W1[:, D_NODE:2 * D_NODE, :]],
                                axis=0)
    table = _project_nodes(node_embed, w_stacked).reshape(
        2 * N_EXPERTS * N_NODES, HIDDEN)

    hs, hd = _sc_gather(table, src, dst, etype_p)

    w1c = W1[:, 2 * D_NODE:, :].transpose(1, 0, 2).reshape(
        D_EDGE, N_EXPERTS * HIDDEN)
    w2c = W2.transpose(1, 0, 2).reshape(HIDDEN, N_EXPERTS * HIDDEN)
    w3c = W3[:, :, 0].T
    t3 = etype.astype(jnp.float32).reshape(N_EDGES, 1)
    bz3 = baseline_z.reshape(N_EDGES, 1)

    dz3, rho3 = _edge_mlp(hs, hd, edge_state, t3, bz3,
                          w1c, b1, w2c, b2, w3c, b3)
    return dz3.reshape(N_EDGES), rho3.reshape(N_EDGES)
